# baseline (speedup 1.0000x reference)
import functools

import jax
import jax.numpy as jnp
from jax.experimental import pallas as pl
from jax.experimental.pallas import tpu as pltpu

_EPS = 1e-5
_N = 64


def _affine(w, b, m, v):
    s = w / jnp.sqrt(v + _EPS)
    t = b - m * s
    return (s.astype(jnp.float32).reshape(1, -1),
            t.astype(jnp.float32).reshape(1, -1))


def _bf(x):
    return x.astype(jnp.bfloat16)


def _bspec(shape):
    return pl.BlockSpec(shape, lambda i, _n=len(shape): (0,) * _n)


def _xspec(shape, B=1):
    return pl.BlockSpec((B,) + tuple(shape[1:]),
                        lambda i, _n=len(shape) - 1: (i,) + (0,) * _n)


_SROWS = 28


def _stem_kernel(x_ref, w_ref, s_ref, t_ref, o_ref):
    k = pl.program_id(1)
    cols = []
    for a in range(4):
        for b in range(4):
            cols.append(x_ref[0, pl.ds(_SROWS * k + a, _SROWS),
                              b:b + 112, :])
    patches = jnp.concatenate(cols, axis=-1).reshape(_SROWS * 112, 192)
    acc = jnp.dot(patches, w_ref[...], preferred_element_type=jnp.float32)
    y = jnp.maximum(acc * s_ref[...] + t_ref[...], 0.0)
    o_ref[0] = y.reshape(_SROWS, 112, 16).astype(jnp.bfloat16)


def _stem(img, conv0, s, t):
    x = _bf(jnp.transpose(img, (0, 2, 3, 1)))
    xp = jnp.pad(x, ((0, 0), (3, 3), (3, 3), (0, 0)))
    x12 = (xp.reshape(_N, 115, 2, 115, 2, 3)
           .transpose(0, 1, 3, 2, 4, 5)
           .reshape(_N, 115, 115, 12))
    w8 = jnp.pad(conv0, ((0, 0), (0, 0), (0, 1), (0, 1)))
    w4 = _bf(w8.reshape(16, 3, 4, 2, 4, 2)
             .transpose(2, 4, 3, 5, 1, 0)
             .reshape(192, 16))
    return pl.pallas_call(
        _stem_kernel,
        out_shape=jax.ShapeDtypeStruct((_N, 112, 112, 16), jnp.bfloat16),
        grid=(_N, 112 // _SROWS),
        in_specs=[pl.BlockSpec((1, 115, 115, 12), lambda i, k: (i, 0, 0, 0)),
                  pl.BlockSpec(w4.shape, lambda i, k: (0, 0)),
                  pl.BlockSpec(s.shape, lambda i, k: (0, 0)),
                  pl.BlockSpec(t.shape, lambda i, k: (0, 0))],
        out_specs=pl.BlockSpec((1, _SROWS, 112, 16),
                               lambda i, k: (i, k, 0, 0)),
        compiler_params=pltpu.CompilerParams(
            dimension_semantics=("parallel", "arbitrary")),
    )(x12, w4, s, t)


def _dense_core(x, p, B, H, W, Cin):
    s1, t1, w1, s2, t2, w2 = p
    t = jnp.maximum(x.astype(jnp.float32) * s1 + t1, 0.0)
    b_ = jnp.dot(_bf(t.reshape(B * H * W, Cin)), w1,
                 preferred_element_type=jnp.float32)
    b_ = jnp.maximum(b_ * s2 + t2, 0.0)
    b3 = _bf(b_).reshape(B, H, W, 32)
    zr = jnp.zeros((B, 1, W, 32), jnp.bfloat16)
    bp = jnp.concatenate([zr, b3, zr], axis=1)
    zc = jnp.zeros((B, H + 2, 1, 32), jnp.bfloat16)
    bp = jnp.concatenate([zc, bp, zc], axis=2)
    acc = jnp.zeros((B * H * W, 8), jnp.float32)
    for ki in range(3):
        for kj in range(3):
            sl = bp[:, ki:ki + H, kj:kj + W, :].reshape(B * H * W, 32)
            acc += jnp.dot(sl, w2[ki, kj],
                           preferred_element_type=jnp.float32)
    return jnp.concatenate([x, _bf(acc).reshape(B, H, W, 8)], axis=-1)


def _transition_1x1(out, p, B, H, W):
    sT, tT, wT = p
    tt = jnp.maximum(out.astype(jnp.float32) * sT + tT, 0.0)
    y = jnp.dot(_bf(tt.reshape(B * H * W, 32)), wT,
                preferred_element_type=jnp.float32)
    return y.reshape(B, H, W, 16)


def _avgpool2(y, B, H, W):
    yh = y.reshape(B, H // 2, 2, W, 16)
    yh = yh[:, :, 0] + yh[:, :, 1]
    r0 = jax.lax.broadcasted_iota(jnp.int32, (W, W // 2), 0)
    r1 = jax.lax.broadcasted_iota(jnp.int32, (W, W // 2), 1)
    pmat = jnp.where(r0 // 2 == r1, 0.25, 0.0)
    z = jnp.transpose(yh.reshape(B * (H // 2), W, 16), (0, 2, 1))
    z = jnp.dot(z.reshape(B * (H // 2) * 16, W), pmat,
                preferred_element_type=jnp.float32)
    z = jnp.transpose(z.reshape(B * (H // 2), 16, W // 2), (0, 2, 1))
    return z.reshape(B, H // 2, W // 2, 16)


def _block0_kernel(x_ref, *refs, B):
    o_ref = refs[-1]
    l0 = tuple(r[...] for r in refs[0:6])
    l1 = tuple(r[...] for r in refs[6:12])
    tr = tuple(r[...] for r in refs[12:15])
    taps = [(0, 0), (1, 0), (0, 1)]
    m = None
    for (pi, ai) in taps:
        for (qj, bj) in taps:
            g = (pi * 2 + qj) * 16
            sl = x_ref[:, ai:ai + 56, bj:bj + 56, g:g + 16]
            m = sl if m is None else jnp.maximum(m, sl)
    out = _dense_core(m, l0, B, 56, 56, 16)
    out = _dense_core(out, l1, B, 56, 56, 24)
    y = _transition_1x1(out, tr, B, 56, 56)
    o_ref[...] = _bf(y)


def _tail_kernel(x_ref, *refs, B):
    o_ref = refs[-1]
    i = 0

    def take(n):
        nonlocal i
        vals = tuple(r[...] for r in refs[i:i + n])
        i += n
        return vals

    x = x_ref[...].astype(jnp.float32)
    hw = 56
    for bi in range(3):
        x = _bf(_avgpool2(x, B, hw, hw))
        hw //= 2
        out = _dense_core(x, take(6), B, hw, hw, 16)
        out = _dense_core(out, take(6), B, hw, hw, 24)
        if bi < 2:
            x = _transition_1x1(out, take(3), B, hw, hw)
    s5, t5, cw, cb = take(4)
    tt = jnp.maximum(out.astype(jnp.float32) * s5 + t5, 0.0)
    feats = jnp.mean(tt.reshape(B, hw * hw, 32), axis=1)
    logits = jnp.sum(feats * cw, axis=-1, keepdims=True) + cb
    o_ref[...] = jnp.broadcast_to(logits[:, :, None], (B, 1, 128))


def _parity4_maxpool(y):
    yp = jnp.pad(y, ((0, 0), (1, 1), (1, 1), (0, 0)),
                 constant_values=jnp.bfloat16(-1e30))
    return (yp.reshape(_N, 57, 2, 57, 2, 16)
            .transpose(0, 1, 3, 2, 4, 5)
            .reshape(_N, 57, 57, 64))


def _layer_params(l):
    bn1 = _affine(*l[0:4])
    bn2 = _affine(*l[5:9])
    return [bn1[0], bn1[1], _bf(jnp.transpose(l[4][:, :, 0, 0])),
            bn2[0], bn2[1], _bf(jnp.transpose(l[9], (2, 3, 1, 0)))]


def _trans_params(t):
    sT, tT = _affine(*t[0:4])
    return [sT, tT, _bf(jnp.transpose(t[4][:, :, 0, 0]))]


def kernel(img, conv0, norm0_weight, norm0_bias, norm0_mean, norm0_var, b0_l0_norm1_weight, b0_l0_norm1_bias, b0_l0_norm1_mean, b0_l0_norm1_var, b0_l0_conv1, b0_l0_norm2_weight, b0_l0_norm2_bias, b0_l0_norm2_mean, b0_l0_norm2_var, b0_l0_conv2, b0_l1_norm1_weight, b0_l1_norm1_bias, b0_l1_norm1_mean, b0_l1_norm1_var, b0_l1_conv1, b0_l1_norm2_weight, b0_l1_norm2_bias, b0_l1_norm2_mean, b0_l1_norm2_var, b0_l1_conv2, b1_l0_norm1_weight, b1_l0_norm1_bias, b1_l0_norm1_mean, b1_l0_norm1_var, b1_l0_conv1, b1_l0_norm2_weight, b1_l0_norm2_bias, b1_l0_norm2_mean, b1_l0_norm2_var, b1_l0_conv2, b1_l1_norm1_weight, b1_l1_norm1_bias, b1_l1_norm1_mean, b1_l1_norm1_var, b1_l1_conv1, b1_l1_norm2_weight, b1_l1_norm2_bias, b1_l1_norm2_mean, b1_l1_norm2_var, b1_l1_conv2, b2_l0_norm1_weight, b2_l0_norm1_bias, b2_l0_norm1_mean, b2_l0_norm1_var, b2_l0_conv1, b2_l0_norm2_weight, b2_l0_norm2_bias, b2_l0_norm2_mean, b2_l0_norm2_var, b2_l0_conv2, b2_l1_norm1_weight, b2_l1_norm1_bias, b2_l1_norm1_mean, b2_l1_norm1_var, b2_l1_conv1, b2_l1_norm2_weight, b2_l1_norm2_bias, b2_l1_norm2_mean, b2_l1_norm2_var, b2_l1_conv2, b3_l0_norm1_weight, b3_l0_norm1_bias, b3_l0_norm1_mean, b3_l0_norm1_var, b3_l0_conv1, b3_l0_norm2_weight, b3_l0_norm2_bias, b3_l0_norm2_mean, b3_l0_norm2_var, b3_l0_conv2, b3_l1_norm1_weight, b3_l1_norm1_bias, b3_l1_norm1_mean, b3_l1_norm1_var, b3_l1_conv1, b3_l1_norm2_weight, b3_l1_norm2_bias, b3_l1_norm2_mean, b3_l1_norm2_var, b3_l1_conv2, t0_norm_weight, t0_norm_bias, t0_norm_mean, t0_norm_var, t0_conv, t1_norm_weight, t1_norm_bias, t1_norm_mean, t1_norm_var, t1_conv, t2_norm_weight, t2_norm_bias, t2_norm_mean, t2_norm_var, t2_conv, norm5_weight, norm5_bias, norm5_mean, norm5_var, classifier_w, classifier_b):
    s0, t0 = _affine(norm0_weight, norm0_bias, norm0_mean, norm0_var)
    y = _stem(img, conv0, s0, t0)
    x = _parity4_maxpool(y)

    layers = [
        (b0_l0_norm1_weight, b0_l0_norm1_bias, b0_l0_norm1_mean, b0_l0_norm1_var,
         b0_l0_conv1, b0_l0_norm2_weight, b0_l0_norm2_bias, b0_l0_norm2_mean,
         b0_l0_norm2_var, b0_l0_conv2),
        (b0_l1_norm1_weight, b0_l1_norm1_bias, b0_l1_norm1_mean, b0_l1_norm1_var,
         b0_l1_conv1, b0_l1_norm2_weight, b0_l1_norm2_bias, b0_l1_norm2_mean,
         b0_l1_norm2_var, b0_l1_conv2),
        (b1_l0_norm1_weight, b1_l0_norm1_bias, b1_l0_norm1_mean, b1_l0_norm1_var,
         b1_l0_conv1, b1_l0_norm2_weight, b1_l0_norm2_bias, b1_l0_norm2_mean,
         b1_l0_norm2_var, b1_l0_conv2),
        (b1_l1_norm1_weight, b1_l1_norm1_bias, b1_l1_norm1_mean, b1_l1_norm1_var,
         b1_l1_conv1, b1_l1_norm2_weight, b1_l1_norm2_bias, b1_l1_norm2_mean,
         b1_l1_norm2_var, b1_l1_conv2),
        (b2_l0_norm1_weight, b2_l0_norm1_bias, b2_l0_norm1_mean, b2_l0_norm1_var,
         b2_l0_conv1, b2_l0_norm2_weight, b2_l0_norm2_bias, b2_l0_norm2_mean,
         b2_l0_norm2_var, b2_l0_conv2),
        (b2_l1_norm1_weight, b2_l1_norm1_bias, b2_l1_norm1_mean, b2_l1_norm1_var,
         b2_l1_conv1, b2_l1_norm2_weight, b2_l1_norm2_bias, b2_l1_norm2_mean,
         b2_l1_norm2_var, b2_l1_conv2),
        (b3_l0_norm1_weight, b3_l0_norm1_bias, b3_l0_norm1_mean, b3_l0_norm1_var,
         b3_l0_conv1, b3_l0_norm2_weight, b3_l0_norm2_bias, b3_l0_norm2_mean,
         b3_l0_norm2_var, b3_l0_conv2),
        (b3_l1_norm1_weight, b3_l1_norm1_bias, b3_l1_norm1_mean, b3_l1_norm1_var,
         b3_l1_conv1, b3_l1_norm2_weight, b3_l1_norm2_bias, b3_l1_norm2_mean,
         b3_l1_norm2_var, b3_l1_conv2),
    ]
    trans = [
        (t0_norm_weight, t0_norm_bias, t0_norm_mean, t0_norm_var, t0_conv),
        (t1_norm_weight, t1_norm_bias, t1_norm_mean, t1_norm_var, t1_conv),
        (t2_norm_weight, t2_norm_bias, t2_norm_mean, t2_norm_var, t2_conv),
    ]

    B0 = 4
    ins0 = ([x] + _layer_params(layers[0]) + _layer_params(layers[1])
            + _trans_params(trans[0]))
    x = pl.pallas_call(
        functools.partial(_block0_kernel, B=B0),
        out_shape=jax.ShapeDtypeStruct((_N, 56, 56, 16), jnp.bfloat16),
        grid=(_N // B0,),
        in_specs=[_xspec(x.shape, B0)] + [_bspec(a.shape) for a in ins0[1:]],
        out_specs=_xspec((_N, 56, 56, 16), B0),
        compiler_params=pltpu.CompilerParams(
            dimension_semantics=("parallel",)),
    )(*ins0)

    BT = 8
    s5, t5 = _affine(norm5_weight, norm5_bias, norm5_mean, norm5_var)
    cw = classifier_w.astype(jnp.float32)
    cb = classifier_b.astype(jnp.float32).reshape(1, 1)
    inst = [x]
    for bi in range(1, 4):
        inst += _layer_params(layers[2 * bi]) + _layer_params(layers[2 * bi + 1])
        if bi < 3:
            inst += _trans_params(trans[bi])
    inst += [s5, t5, cw, cb]
    out = pl.pallas_call(
        functools.partial(_tail_kernel, B=BT),
        out_shape=jax.ShapeDtypeStruct((_N, 1, 128), jnp.float32),
        grid=(_N // BT,),
        in_specs=[_xspec(x.shape, BT)] + [_bspec(a.shape) for a in inst[1:]],
        out_specs=pl.BlockSpec((BT, 1, 128), lambda i: (i, 0, 0)),
        compiler_params=pltpu.CompilerParams(
            dimension_semantics=("parallel",)),
    )(*inst)
    return out[:, 0, :1]


# speedup vs baseline: 1.1519x; 1.1519x over previous
import functools

import jax
import jax.numpy as jnp
from jax.experimental import pallas as pl
from jax.experimental.pallas import tpu as pltpu

_EPS = 1e-5
_N = 64


def _affine(w, b, m, v):
    s = w / jnp.sqrt(v + _EPS)
    t = b - m * s
    return (s.astype(jnp.float32).reshape(1, -1),
            t.astype(jnp.float32).reshape(1, -1))


def _bf(x):
    return x.astype(jnp.bfloat16)


def _bspec(shape):
    return pl.BlockSpec(shape, lambda i, _n=len(shape): (0,) * _n)


def _xspec(shape, B=1):
    return pl.BlockSpec((B,) + tuple(shape[1:]),
                        lambda i, _n=len(shape) - 1: (i,) + (0,) * _n)


_SROWS = 28


def _stem_kernel(x_ref, w_ref, s_ref, t_ref, o_ref):
    k = pl.program_id(1)
    cols = []
    for a in range(4):
        for b in range(4):
            cols.append(x_ref[0, pl.ds(_SROWS * k + a, _SROWS),
                              b:b + 112, :])
    patches = jnp.concatenate(cols, axis=-1).reshape(_SROWS * 112, 192)
    acc = jnp.dot(patches, w_ref[...], preferred_element_type=jnp.float32)
    y = jnp.maximum(acc * s_ref[...] + t_ref[...], 0.0)
    o_ref[0] = y.reshape(_SROWS, 112, 16).astype(jnp.bfloat16)


def _stem(img, conv0, s, t):
    x = _bf(jnp.transpose(img, (0, 2, 3, 1)))
    xp = jnp.pad(x, ((0, 0), (3, 3), (3, 3), (0, 0)))
    x12 = (xp.reshape(_N, 115, 2, 115, 2, 3)
           .transpose(0, 1, 3, 2, 4, 5)
           .reshape(_N, 115, 115, 12))
    w8 = jnp.pad(conv0, ((0, 0), (0, 0), (0, 1), (0, 1)))
    w4 = _bf(w8.reshape(16, 3, 4, 2, 4, 2)
             .transpose(2, 4, 3, 5, 1, 0)
             .reshape(192, 16))
    return pl.pallas_call(
        _stem_kernel,
        out_shape=jax.ShapeDtypeStruct((_N, 112, 112, 16), jnp.bfloat16),
        grid=(_N, 112 // _SROWS),
        in_specs=[pl.BlockSpec((1, 115, 115, 12), lambda i, k: (i, 0, 0, 0)),
                  pl.BlockSpec(w4.shape, lambda i, k: (0, 0)),
                  pl.BlockSpec(s.shape, lambda i, k: (0, 0)),
                  pl.BlockSpec(t.shape, lambda i, k: (0, 0))],
        out_specs=pl.BlockSpec((1, _SROWS, 112, 16),
                               lambda i, k: (i, k, 0, 0)),
        compiler_params=pltpu.CompilerParams(
            dimension_semantics=("parallel", "arbitrary")),
    )(x12, w4, s, t)


def _layer_kernel(*refs, B, H, W, Cin, pre, tail):
    o_ref = refs[-1]
    x_ref, s1_ref, t1_ref, w1_ref, s2_ref, t2_ref, w2_ref = refs[:7]
    extra = refs[7:-1]

    if pre == "max":
        taps = [(0, 0), (1, 0), (0, 1)]
        m = None
        for (pi, ai) in taps:
            for (qj, bj) in taps:
                g = (pi * 2 + qj) * Cin
                sl = x_ref[:, ai:ai + H, bj:bj + W, g:g + Cin]
                m = sl if m is None else jnp.maximum(m, sl)
        x = m
    elif pre == "avg":
        xx = x_ref[...]
        x = _bf((xx[..., 0 * Cin:1 * Cin].astype(jnp.float32)
                 + xx[..., 1 * Cin:2 * Cin] + xx[..., 2 * Cin:3 * Cin]
                 + xx[..., 3 * Cin:4 * Cin]) * 0.25)
    else:
        x = x_ref[...]

    t = jnp.maximum(x.astype(jnp.float32) * s1_ref[...] + t1_ref[...], 0.0)
    b_ = jnp.dot(_bf(t.reshape(B * H * W, Cin)), w1_ref[...],
                 preferred_element_type=jnp.float32)
    b_ = jnp.maximum(b_ * s2_ref[...] + t2_ref[...], 0.0)
    b3 = _bf(b_).reshape(B, H, W, 32)
    zr = jnp.zeros((B, 1, W, 32), jnp.bfloat16)
    bp = jnp.concatenate([zr, b3, zr], axis=1)
    zc = jnp.zeros((B, H + 2, 1, 32), jnp.bfloat16)
    bp = jnp.concatenate([zc, bp, zc], axis=2)
    acc = jnp.zeros((B * H * W, 8), jnp.float32)
    for ki in range(3):
        for kj in range(3):
            sl = bp[:, ki:ki + H, kj:kj + W, :].reshape(B * H * W, 32)
            acc += jnp.dot(sl, w2_ref[ki, kj],
                           preferred_element_type=jnp.float32)
    out = jnp.concatenate([x, _bf(acc).reshape(B, H, W, 8)], axis=-1)

    if tail == "trans":
        sT_ref, tT_ref, wT_ref = extra
        tt = jnp.maximum(out.astype(jnp.float32) * sT_ref[...] + tT_ref[...],
                         0.0)
        y = jnp.dot(_bf(tt.reshape(B * H * W, Cin + 8)), wT_ref[...],
                    preferred_element_type=jnp.float32)
        o_ref[...] = _bf(y).reshape(B, H, W, 16)
    elif tail == "head":
        s5_ref, t5_ref, cw_ref, cb_ref = extra
        tt = jnp.maximum(out.astype(jnp.float32) * s5_ref[...] + t5_ref[...],
                         0.0)
        feats = jnp.mean(tt.reshape(B, H * W, Cin + 8), axis=1)
        logits = (jnp.sum(feats * cw_ref[...], axis=-1, keepdims=True)
                  + cb_ref[...])
        o_ref[...] = jnp.broadcast_to(logits[:, :, None], (B, 1, 128))
    else:
        o_ref[...] = out


def _layer(x, bn1, w1, bn2, w2, B, H, W, Cin, pre, tail, extra):
    ins = [x, bn1[0], bn1[1], _bf(jnp.transpose(w1[:, :, 0, 0])),
           bn2[0], bn2[1], _bf(jnp.transpose(w2, (2, 3, 1, 0)))] + extra
    if tail == "head":
        out_shape = jax.ShapeDtypeStruct((_N, 1, 128), jnp.float32)
        out_spec = pl.BlockSpec((B, 1, 128), lambda i: (i, 0, 0))
    elif tail == "trans":
        out_shape = jax.ShapeDtypeStruct((_N, H, W, 16), jnp.bfloat16)
        out_spec = _xspec((_N, H, W, 16), B)
    else:
        out_shape = jax.ShapeDtypeStruct((_N, H, W, Cin + 8), jnp.bfloat16)
        out_spec = _xspec((_N, H, W, Cin + 8), B)
    specs = [_xspec(ins[0].shape, B)] + [_bspec(a.shape) for a in ins[1:]]
    return pl.pallas_call(
        functools.partial(_layer_kernel, B=B, H=H, W=W, Cin=Cin,
                          pre=pre, tail=tail),
        out_shape=out_shape,
        grid=(_N // B,),
        in_specs=specs,
        out_specs=out_spec,
        compiler_params=pltpu.CompilerParams(
            dimension_semantics=("parallel",)),
    )(*ins)


def _parity4(x, pad=0, pad_value=0.0):
    n, h2, w2, c = x.shape
    if pad:
        x = jnp.pad(x, ((0, 0), (pad, pad), (pad, pad), (0, 0)),
                    constant_values=jnp.asarray(pad_value, x.dtype))
        h2, w2 = h2 + 2 * pad, w2 + 2 * pad
    return (x.reshape(n, h2 // 2, 2, w2 // 2, 2, c)
            .transpose(0, 1, 3, 2, 4, 5)
            .reshape(n, h2 // 2, w2 // 2, 4 * c))


def kernel(img, conv0, norm0_weight, norm0_bias, norm0_mean, norm0_var, b0_l0_norm1_weight, b0_l0_norm1_bias, b0_l0_norm1_mean, b0_l0_norm1_var, b0_l0_conv1, b0_l0_norm2_weight, b0_l0_norm2_bias, b0_l0_norm2_mean, b0_l0_norm2_var, b0_l0_conv2, b0_l1_norm1_weight, b0_l1_norm1_bias, b0_l1_norm1_mean, b0_l1_norm1_var, b0_l1_conv1, b0_l1_norm2_weight, b0_l1_norm2_bias, b0_l1_norm2_mean, b0_l1_norm2_var, b0_l1_conv2, b1_l0_norm1_weight, b1_l0_norm1_bias, b1_l0_norm1_mean, b1_l0_norm1_var, b1_l0_conv1, b1_l0_norm2_weight, b1_l0_norm2_bias, b1_l0_norm2_mean, b1_l0_norm2_var, b1_l0_conv2, b1_l1_norm1_weight, b1_l1_norm1_bias, b1_l1_norm1_mean, b1_l1_norm1_var, b1_l1_conv1, b1_l1_norm2_weight, b1_l1_norm2_bias, b1_l1_norm2_mean, b1_l1_norm2_var, b1_l1_conv2, b2_l0_norm1_weight, b2_l0_norm1_bias, b2_l0_norm1_mean, b2_l0_norm1_var, b2_l0_conv1, b2_l0_norm2_weight, b2_l0_norm2_bias, b2_l0_norm2_mean, b2_l0_norm2_var, b2_l0_conv2, b2_l1_norm1_weight, b2_l1_norm1_bias, b2_l1_norm1_mean, b2_l1_norm1_var, b2_l1_conv1, b2_l1_norm2_weight, b2_l1_norm2_bias, b2_l1_norm2_mean, b2_l1_norm2_var, b2_l1_conv2, b3_l0_norm1_weight, b3_l0_norm1_bias, b3_l0_norm1_mean, b3_l0_norm1_var, b3_l0_conv1, b3_l0_norm2_weight, b3_l0_norm2_bias, b3_l0_norm2_mean, b3_l0_norm2_var, b3_l0_conv2, b3_l1_norm1_weight, b3_l1_norm1_bias, b3_l1_norm1_mean, b3_l1_norm1_var, b3_l1_conv1, b3_l1_norm2_weight, b3_l1_norm2_bias, b3_l1_norm2_mean, b3_l1_norm2_var, b3_l1_conv2, t0_norm_weight, t0_norm_bias, t0_norm_mean, t0_norm_var, t0_conv, t1_norm_weight, t1_norm_bias, t1_norm_mean, t1_norm_var, t1_conv, t2_norm_weight, t2_norm_bias, t2_norm_mean, t2_norm_var, t2_conv, norm5_weight, norm5_bias, norm5_mean, norm5_var, classifier_w, classifier_b):
    s0, t0 = _affine(norm0_weight, norm0_bias, norm0_mean, norm0_var)
    y = _stem(img, conv0, s0, t0)
    x = _parity4(y, pad=1, pad_value=-1e30)

    layers = [
        (b0_l0_norm1_weight, b0_l0_norm1_bias, b0_l0_norm1_mean, b0_l0_norm1_var,
         b0_l0_conv1, b0_l0_norm2_weight, b0_l0_norm2_bias, b0_l0_norm2_mean,
         b0_l0_norm2_var, b0_l0_conv2),
        (b0_l1_norm1_weight, b0_l1_norm1_bias, b0_l1_norm1_mean, b0_l1_norm1_var,
         b0_l1_conv1, b0_l1_norm2_weight, b0_l1_norm2_bias, b0_l1_norm2_mean,
         b0_l1_norm2_var, b0_l1_conv2),
        (b1_l0_norm1_weight, b1_l0_norm1_bias, b1_l0_norm1_mean, b1_l0_norm1_var,
         b1_l0_conv1, b1_l0_norm2_weight, b1_l0_norm2_bias, b1_l0_norm2_mean,
         b1_l0_norm2_var, b1_l0_conv2),
        (b1_l1_norm1_weight, b1_l1_norm1_bias, b1_l1_norm1_mean, b1_l1_norm1_var,
         b1_l1_conv1, b1_l1_norm2_weight, b1_l1_norm2_bias, b1_l1_norm2_mean,
         b1_l1_norm2_var, b1_l1_conv2),
        (b2_l0_norm1_weight, b2_l0_norm1_bias, b2_l0_norm1_mean, b2_l0_norm1_var,
         b2_l0_conv1, b2_l0_norm2_weight, b2_l0_norm2_bias, b2_l0_norm2_mean,
         b2_l0_norm2_var, b2_l0_conv2),
        (b2_l1_norm1_weight, b2_l1_norm1_bias, b2_l1_norm1_mean, b2_l1_norm1_var,
         b2_l1_conv1, b2_l1_norm2_weight, b2_l1_norm2_bias, b2_l1_norm2_mean,
         b2_l1_norm2_var, b2_l1_conv2),
        (b3_l0_norm1_weight, b3_l0_norm1_bias, b3_l0_norm1_mean, b3_l0_norm1_var,
         b3_l0_conv1, b3_l0_norm2_weight, b3_l0_norm2_bias, b3_l0_norm2_mean,
         b3_l0_norm2_var, b3_l0_conv2),
        (b3_l1_norm1_weight, b3_l1_norm1_bias, b3_l1_norm1_mean, b3_l1_norm1_var,
         b3_l1_conv1, b3_l1_norm2_weight, b3_l1_norm2_bias, b3_l1_norm2_mean,
         b3_l1_norm2_var, b3_l1_conv2),
    ]
    trans = [
        (t0_norm_weight, t0_norm_bias, t0_norm_mean, t0_norm_var, t0_conv),
        (t1_norm_weight, t1_norm_bias, t1_norm_mean, t1_norm_var, t1_conv),
        (t2_norm_weight, t2_norm_bias, t2_norm_mean, t2_norm_var, t2_conv),
    ]
    hw = [56, 28, 14, 7]
    for bi in range(4):
        H = hw[bi]
        l0, l1 = layers[2 * bi], layers[2 * bi + 1]
        pre0 = "max" if bi == 0 else "avg"
        bn1 = _affine(*l0[0:4])
        bn2 = _affine(*l0[5:9])
        B = (4, 8, 16, 16)[bi]
        x = _layer(x, bn1, l0[4], bn2, l0[9], B, H, H, 16, pre0, "plain", [])
        bn1 = _affine(*l1[0:4])
        bn2 = _affine(*l1[5:9])
        if bi < 3:
            sT, tT = _affine(*trans[bi][0:4])
            wT = _bf(jnp.transpose(trans[bi][4][:, :, 0, 0]))
            x = _layer(x, bn1, l1[4], bn2, l1[9], B, H, H, 24, "plain",
                       "trans", [sT, tT, wT])
            x = _parity4(x)
        else:
            s5, t5 = _affine(norm5_weight, norm5_bias, norm5_mean, norm5_var)
            cw = classifier_w.astype(jnp.float32)
            cb = classifier_b.astype(jnp.float32).reshape(1, 1)
            x = _layer(x, bn1, l1[4], bn2, l1[9], B, H, H, 24, "plain",
                       "head", [s5, t5, cw, cb])
    return x[:, 0, :1]
